# Initial kernel scaffold; baseline (speedup 1.0000x reference)
import jax
import jax.numpy as jnp
from jax.experimental import pallas as pl


def kernel(x, adjacency_list, degree_list, W1s, W1n, b1, g1, be1, W2s, W2n, b2, g2, be2, Wd, bd):
    raise NotImplementedError("write your pallas kernel here")


# speedup vs baseline: 7.5022x; 7.5022x over previous
import functools

import jax
import jax.numpy as jnp
from jax import lax
from jax.experimental import pallas as pl
from jax.experimental.pallas import tpu as pltpu
from jax.experimental.pallas import tpu_sc as plsc

N = 10000
E = 320000
D = 128
H = 128
T = 12

NC = 2
NS = 16
NW = NC * NS
EPW = E // NW
K = 80
NCHUNK = EPW // K
RPT = N // NS
ZR = 125


def _make_segsum():
    mesh = plsc.VectorSubcoreMesh(core_axis_name="c", subcore_axis_name="s")

    @functools.partial(
        pl.kernel,
        out_type=jax.ShapeDtypeStruct((NC * N, H), jnp.float32),
        mesh=mesh,
        scratch_types=[
            pltpu.VMEM((NCHUNK, K), jnp.int32),
            pltpu.VMEM((NCHUNK, K), jnp.int32),
            pltpu.VMEM((K, H), jnp.float32),
            pltpu.VMEM((ZR, H), jnp.float32),
            pltpu.VMEM_SHARED((N, H), jnp.float32),
            pltpu.SemaphoreType.DMA,
        ],
        compiler_params=pltpu.CompilerParams(use_tc_tiling_on_sc=False),
    )
    def segsum(h_hbm, src_hbm, dst_hbm, out_hbm, src_v, dst_v, rows_v,
               zero_v, acc_sh, sem):
        c = lax.axis_index("c")
        s = lax.axis_index("s")
        wid = c * NS + s

        pltpu.sync_copy(src_hbm.at[pl.ds(wid * NCHUNK, NCHUNK)], src_v)
        pltpu.sync_copy(dst_hbm.at[pl.ds(wid * NCHUNK, NCHUNK)], dst_v)

        zero16 = jnp.zeros((16,), jnp.float32)

        def zbody(i, carry):
            for j in range(H // 16):
                zero_v[i, pl.ds(j * 16, 16)] = zero16
            return carry

        lax.fori_loop(0, ZR, zbody, 0)
        for k in range(RPT // ZR):
            pltpu.sync_copy(zero_v, acc_sh.at[pl.ds(s * RPT + k * ZR, ZR)])
        plsc.subcore_barrier()

        def ebody(i, carry):
            pltpu.async_copy(h_hbm.at[src_v.at[i]], rows_v, sem).wait()
            pltpu.sync_copy(rows_v, acc_sh.at[dst_v.at[i]], add=True)
            return carry

        lax.fori_loop(0, NCHUNK, ebody, 0)
        plsc.subcore_barrier()

        pltpu.sync_copy(acc_sh.at[pl.ds(s * RPT, RPT)],
                        out_hbm.at[pl.ds(c * N + s * RPT, RPT)])

    return segsum


_segsum = _make_segsum()


def _tc_layer_body(x_ref, parts_ref, deg_ref, ws_ref, wn_ref, b_ref, g_ref,
                   be_ref, out_ref):
    nrm = 1.0 / jnp.maximum(deg_ref[...].astype(jnp.float32), 1.0)
    agg = (parts_ref[:N, :] + parts_ref[N:, :]) * nrm
    h = (jnp.dot(x_ref[...], ws_ref[...], preferred_element_type=jnp.float32)
         + jnp.dot(agg, wn_ref[...], preferred_element_type=jnp.float32)
         + b_ref[...])
    m = jnp.mean(h, axis=0, keepdims=True)
    v = jnp.mean((h - m) ** 2, axis=0, keepdims=True)
    hn = (h - m) * lax.rsqrt(v + 1e-5) * g_ref[...] + be_ref[...]
    out_ref[...] = jnp.maximum(hn, 0.0)


def _tc_final_body(x_ref, parts_ref, deg_ref, ws_ref, wn_ref, b_ref, g_ref,
                   be_ref, wd_ref, bd_ref, out_ref):
    nrm = 1.0 / jnp.maximum(deg_ref[...].astype(jnp.float32), 1.0)
    agg = (parts_ref[:N, :] + parts_ref[N:, :]) * nrm
    h = (jnp.dot(x_ref[...], ws_ref[...], preferred_element_type=jnp.float32)
         + jnp.dot(agg, wn_ref[...], preferred_element_type=jnp.float32)
         + b_ref[...])
    m = jnp.mean(h, axis=0, keepdims=True)
    v = jnp.mean((h - m) ** 2, axis=0, keepdims=True)
    hn = (h - m) * lax.rsqrt(v + 1e-5) * g_ref[...] + be_ref[...]
    h2 = jnp.maximum(hn, 0.0)
    pooled = jnp.mean(h2, axis=0, keepdims=True)
    out_ref[...] = (jnp.dot(pooled, wd_ref[...],
                            preferred_element_type=jnp.float32) + bd_ref[...])


def _tc_layer(x, parts, deg, ws, wn, b, g, be):
    return pl.pallas_call(
        _tc_layer_body,
        out_shape=jax.ShapeDtypeStruct((N, H), jnp.float32),
    )(x, parts, deg, ws, wn, b.reshape(1, H), g.reshape(1, H),
      be.reshape(1, H))


def _tc_final(x, parts, deg, ws, wn, b, g, be, wd, bd):
    return pl.pallas_call(
        _tc_final_body,
        out_shape=jax.ShapeDtypeStruct((1, T), jnp.float32),
    )(x, parts, deg, ws, wn, b.reshape(1, H), g.reshape(1, H),
      be.reshape(1, H), wd, bd.reshape(1, T))


def kernel(x, adjacency_list, degree_list, W1s, W1n, b1, g1, be1, W2s, W2n,
           b2, g2, be2, Wd, bd):
    src = adjacency_list[0].reshape(E // K, K)
    dst = adjacency_list[1].reshape(E // K, K)
    deg = degree_list.reshape(N, 1)

    parts1 = _segsum(x, src, dst)
    h1 = _tc_layer(x, parts1, deg, W1s, W1n, b1, g1, be1)
    parts2 = _segsum(h1, src, dst)
    out = _tc_final(h1, parts2, deg, W2s, W2n, b2, g2, be2, Wd, bd)
    return out.reshape(T)
